# baseline (speedup 1.0000x reference)
import functools

import jax
import jax.numpy as jnp
from jax import lax
from jax.experimental import pallas as pl
from jax.experimental.pallas import tpu as pltpu
from jax.experimental.pallas import tpu_sc as plsc

_N = 16 * 32 * 32
_NW = 32
_CHUNK = _N // _NW
_OUT_CHUNK = 2 * _CHUNK
_LANES = 16
_THRESH = 0.5


def _gate_body(ent_hbm, out_hbm, ent_v, out_v):
    wid = lax.axis_index("s") * 2 + lax.axis_index("c")
    pltpu.sync_copy(ent_hbm.at[pl.ds(wid * _CHUNK, _CHUNK)], ent_v)
    lane = lax.broadcasted_iota(jnp.int32, (_LANES,), 0)
    for j in range(_CHUNK // _LANES):
        e = ent_v[pl.ds(j * _LANES, _LANES)]
        fine = jnp.where(e > _THRESH, jnp.int32(1), jnp.int32(0))
        coarse = jnp.int32(1) - fine
        idx = j * (2 * _LANES) + 2 * lane
        plsc.store_scatter(out_v, [idx], coarse)
        plsc.store_scatter(out_v, [idx + 1], fine)
    pltpu.sync_copy(out_v, out_hbm.at[pl.ds(wid * _OUT_CHUNK, _OUT_CHUNK)])


@jax.jit
def kernel(h_fine, h_coarse, entropy):
    del h_fine, h_coarse
    ent_flat = entropy.reshape(_N)
    gate_flat = pl.kernel(
        _gate_body,
        mesh=plsc.VectorSubcoreMesh(core_axis_name="c", subcore_axis_name="s"),
        out_type=jax.ShapeDtypeStruct((2 * _N,), jnp.int32),
        scratch_types=[
            pltpu.VMEM((_CHUNK,), jnp.float32),
            pltpu.VMEM((_OUT_CHUNK,), jnp.int32),
        ],
        compiler_params=pltpu.CompilerParams(needs_layout_passes=False),
    )(ent_flat)
    return gate_flat.reshape(16, 32, 32, 2)


# speedup vs baseline: 20.4290x; 20.4290x over previous
import jax
import jax.numpy as jnp
from jax.experimental import pallas as pl
from jax.experimental.pallas import tpu as pltpu

_THRESH = 0.5


def _gate_body(ent_ref, out_ref):
    m = jnp.where(ent_ref[...] > _THRESH, jnp.int32(65536), jnp.int32(1))
    pairs = pltpu.bitcast(m, jnp.int16)
    out_ref[...] = pairs.astype(jnp.int32)


@jax.jit
def kernel(h_fine, h_coarse, entropy):
    del h_fine, h_coarse
    rows = pl.pallas_call(
        _gate_body,
        out_shape=jax.ShapeDtypeStruct((16, 64, 32), jnp.int32),
    )(entropy)
    return rows.reshape(16, 32, 2, 32).transpose(0, 1, 3, 2)
